# baseline (speedup 1.0000x reference)
import functools

import jax
import jax.numpy as jnp
import numpy as np
from jax import lax
from jax.experimental import pallas as pl
from jax.experimental.pallas import tpu as pltpu
from jax.experimental.pallas import tpu_sc as plsc

DEGREE = 3
LANES = 16


@functools.cache
def _spline_sc(num_segments, n_t):
    info = plsc.get_sparse_core_info()
    nc, ns = info.num_cores, info.num_subcores
    nw = nc * ns
    chunk = n_t // nw
    NB = 4
    SUB = chunk // NB
    n_cp = num_segments * (DEGREE - 1) * 2
    n_jp = (num_segments + 1) * 2

    @functools.partial(
        pl.kernel,
        out_type=jax.ShapeDtypeStruct((2, n_t), jnp.float32),
        mesh=plsc.VectorSubcoreMesh(core_axis_name="c", subcore_axis_name="s"),
        compiler_params=pltpu.CompilerParams(needs_layout_passes=False),
        scratch_types=[
            pltpu.VMEM((SUB,), jnp.float32),
            pltpu.VMEM((SUB,), jnp.float32),
            pltpu.VMEM((SUB,), jnp.float32),
            pltpu.VMEM((SUB,), jnp.float32),
            pltpu.VMEM((SUB,), jnp.float32),
            pltpu.VMEM((SUB,), jnp.float32),
            pltpu.VMEM((8 * num_segments,), jnp.float32),
            pltpu.VMEM((n_cp,), jnp.float32),
            pltpu.VMEM((n_jp,), jnp.float32),
            pltpu.SemaphoreType.DMA,
            pltpu.SemaphoreType.DMA,
            pltpu.SemaphoreType.DMA,
            pltpu.SemaphoreType.DMA,
        ],
    )
    def k(t_hbm, cp_hbm, jp_hbm, out_hbm,
          t0_v, t1_v, x0_v, x1_v, y0_v, y1_v, tab_v, cp_v, jp_v,
          in0_s, in1_s, out0_s, out1_s):
        wid = lax.axis_index("s") * nc + lax.axis_index("c")
        base = wid * chunk
        t_bufs = (t0_v, t1_v)
        x_bufs = (x0_v, x1_v)
        y_bufs = (y0_v, y1_v)
        in_sems = (in0_s, in1_s)
        out_sems = (out0_s, out1_s)

        def in_copy(j, b):
            return pltpu.make_async_copy(
                t_hbm.at[pl.ds(base + j * SUB, SUB)], t_bufs[b], in_sems[b]
            )

        def out_copy(j, b, row, buf):
            return pltpu.make_async_copy(
                buf, out_hbm.at[row, pl.ds(base + j * SUB, SUB)], out_sems[b]
            )

        in_copy(0, 0).start()
        pltpu.sync_copy(cp_hbm, cp_v)
        pltpu.sync_copy(jp_hbm, jp_v)
        iv = lax.iota(jnp.int32, LANES)
        nseg_f = float(num_segments)
        tmax = float(np.nextafter(np.float32(1.0), np.float32(0.0)))

        @plsc.parallel_loop(0, num_segments, LANES)
        def build(s0):
            segv = s0 + iv
            j2 = segv * 2
            c4 = segv * 4
            s8 = segv * 8

            def jp(off):
                return plsc.load_gather(jp_v, [j2 + off])

            def cp(off):
                return plsc.load_gather(cp_v, [c4 + off])

            plsc.store_scatter(tab_v, [s8], jp(0))
            plsc.store_scatter(tab_v, [s8 + 1], jp(1))
            plsc.store_scatter(tab_v, [s8 + 2], cp(0))
            plsc.store_scatter(tab_v, [s8 + 3], cp(1))
            plsc.store_scatter(tab_v, [s8 + 4], cp(2))
            plsc.store_scatter(tab_v, [s8 + 5], cp(3))
            plsc.store_scatter(tab_v, [s8 + 6], jp(2))
            plsc.store_scatter(tab_v, [s8 + 7], jp(3))

        def compute(t_v, x_v, y_v):
            @plsc.parallel_loop(0, SUB, LANES, unroll=8)
            def body(o):
                tv = t_v[pl.ds(o, LANES)]
                u = jnp.minimum(tv, tmax) * nseg_f
                seg = u.astype(jnp.int32)
                pt = u - seg.astype(jnp.float32)
                seg8 = seg << 3
                omt = 1.0 - pt
                o2 = omt * omt
                p2 = pt * pt
                p3t = 3.0 * pt
                b0 = o2 * omt
                b1 = p3t * o2
                b2 = 3.0 * p2 * omt
                b3 = p2 * pt

                def g(row):
                    return plsc.load_gather(tab_v, [seg8 + row])

                x = b0 * g(0) + b1 * g(2) + b2 * g(4) + b3 * g(6)
                y = b0 * g(1) + b1 * g(3) + b2 * g(5) + b3 * g(7)
                x_v[pl.ds(o, LANES)] = x
                y_v[pl.ds(o, LANES)] = y

        for j in range(NB):
            b = j % 2
            in_copy(j, b).wait()
            if j + 1 < NB:
                in_copy(j + 1, 1 - b).start()
            if j >= 2:
                out_copy(j - 2, b, 0, x_bufs[b]).wait()
                out_copy(j - 2, b, 1, y_bufs[b]).wait()
            compute(t_bufs[b], x_bufs[b], y_bufs[b])
            out_copy(j, b, 0, x_bufs[b]).start()
            out_copy(j, b, 1, y_bufs[b]).start()
        for j in (NB - 2, NB - 1):
            b = j % 2
            out_copy(j, b, 0, x_bufs[b]).wait()
            out_copy(j, b, 1, y_bufs[b]).wait()

    return k


def kernel(t, control_points, joint_points):
    ns = joint_points.shape[0] - 1
    n_t = t.shape[0]
    out = _spline_sc(ns, n_t)(
        t, control_points.reshape(-1), joint_points.reshape(-1)
    )
    return out.T


# speedup vs baseline: 1.5470x; 1.5470x over previous
import functools

import jax
import jax.numpy as jnp
import numpy as np
from jax import lax
from jax.experimental import pallas as pl
from jax.experimental.pallas import tpu as pltpu
from jax.experimental.pallas import tpu_sc as plsc

DEGREE = 3
LANES = 16


@functools.cache
def _spline_sc(num_segments, n_t):
    info = plsc.get_sparse_core_info()
    nc, ns = info.num_cores, info.num_subcores
    nw = nc * ns
    chunk = n_t // nw
    NB = 4
    SUB = chunk // NB
    n_cp = num_segments * (DEGREE - 1) * 2
    n_jp = (num_segments + 1) * 2

    @functools.partial(
        pl.kernel,
        out_type=jax.ShapeDtypeStruct((2, n_t), jnp.float32),
        mesh=plsc.VectorSubcoreMesh(core_axis_name="c", subcore_axis_name="s"),
        compiler_params=pltpu.CompilerParams(needs_layout_passes=False),
        scratch_types=[
            pltpu.VMEM((SUB,), jnp.float32),
            pltpu.VMEM((SUB,), jnp.float32),
            pltpu.VMEM((SUB,), jnp.float32),
            pltpu.VMEM((SUB,), jnp.float32),
            pltpu.VMEM((SUB,), jnp.float32),
            pltpu.VMEM((SUB,), jnp.float32),
            pltpu.VMEM((8 * num_segments + LANES,), jnp.float32),
            pltpu.VMEM((num_segments,), jnp.int32),
            pltpu.VMEM((n_cp,), jnp.float32),
            pltpu.VMEM((n_jp,), jnp.float32),
            pltpu.SemaphoreType.DMA,
            pltpu.SemaphoreType.DMA,
            pltpu.SemaphoreType.DMA,
            pltpu.SemaphoreType.DMA,
        ],
    )
    def k(t_hbm, cp_hbm, jp_hbm, out_hbm,
          t0_v, t1_v, x0_v, x1_v, y0_v, y1_v, ctab_v, thr_v, cp_v, jp_v,
          in0_s, in1_s, out0_s, out1_s):
        wid = lax.axis_index("s") * nc + lax.axis_index("c")
        base = wid * chunk
        t_bufs = (t0_v, t1_v)
        x_bufs = (x0_v, x1_v)
        y_bufs = (y0_v, y1_v)
        in_sems = (in0_s, in1_s)
        out_sems = (out0_s, out1_s)

        def in_copy(j, b):
            return pltpu.make_async_copy(
                t_hbm.at[pl.ds(base + j * SUB, SUB)], t_bufs[b], in_sems[b]
            )

        def out_copy(j, b, row, buf):
            return pltpu.make_async_copy(
                buf, out_hbm.at[row, pl.ds(base + j * SUB, SUB)], out_sems[b]
            )

        in_copy(0, 0).start()
        pltpu.sync_copy(cp_hbm, cp_v)
        pltpu.sync_copy(jp_hbm, jp_v)
        iv = lax.iota(jnp.int32, LANES)
        nseg_f = float(num_segments)
        inv_nseg = 1.0 / num_segments
        tmax = float(np.nextafter(np.float32(1.0), np.float32(0.0)))

        @plsc.parallel_loop(0, num_segments, LANES)
        def build(s0):
            segv = s0 + iv
            j2 = segv * 2
            c4 = segv * 4
            s8 = segv * 8

            def jp(off):
                return plsc.load_gather(jp_v, [j2 + off])

            def cp(off):
                return plsc.load_gather(cp_v, [c4 + off])

            for d in range(2):
                p0 = jp(d)
                p1 = cp(d)
                p2 = cp(2 + d)
                p3 = jp(2 + d)
                c1 = 3.0 * (p1 - p0)
                c2 = 3.0 * ((p0 - p1) + (p2 - p1))
                c3 = (p3 - p0) + 3.0 * (p1 - p2)
                plsc.store_scatter(ctab_v, [s8 + 4 * d], p0)
                plsc.store_scatter(ctab_v, [s8 + 4 * d + 1], c1)
                plsc.store_scatter(ctab_v, [s8 + 4 * d + 2], c2)
                plsc.store_scatter(ctab_v, [s8 + 4 * d + 3], c3)
            thr_f = (segv + 1).astype(jnp.float32) * inv_nseg
            plsc.store_scatter(thr_v, [segv], plsc.bitcast(thr_f, jnp.int32))

        NV = SUB // LANES

        def compute(t_v, x_v, y_v):
            def gather_fix(lo, hi):
                @plsc.parallel_loop(lo, hi, LANES)
                def fix(o):
                    tv = t_v[pl.ds(o, LANES)]
                    u = jnp.minimum(tv, tmax) * nseg_f
                    seg = u.astype(jnp.int32)
                    pt = u - seg.astype(jnp.float32)
                    seg8 = seg << 3

                    def g(r):
                        return plsc.load_gather(ctab_v, [seg8 + r])

                    x = ((g(3) * pt + g(2)) * pt + g(1)) * pt + g(0)
                    y = ((g(7) * pt + g(6)) * pt + g(5)) * pt + g(4)
                    x_v[pl.ds(o, LANES)] = x
                    y_v[pl.ds(o, LANES)] = y

            def lane0_bits(i16):
                v = t_v[pl.ds(i16, LANES)]
                return lax.bitcast_convert_type(v[0], jnp.int32)

            def while_cond(istart):
                return istart < SUB

            def while_body(istart):
                tv0 = t_v[pl.ds(istart, LANES)]
                u0 = jnp.minimum(tv0, tmax) * nseg_f
                seg0 = u0.astype(jnp.int32)
                s_spl = plsc.cummax(
                    jnp.where(iv == 0, seg0, jnp.int32(-2147483648))
                )
                thr_b = plsc.load_gather(thr_v, [s_spl])[0]

                def bs_cond(c):
                    return c[0] < c[1]

                def bs_body(c):
                    lo, hi = c
                    mid = (lo + hi) >> 1
                    pred = lane0_bits(mid * LANES) < thr_b
                    return (jnp.where(pred, mid + 1, lo),
                            jnp.where(pred, hi, mid))

                vb = lax.while_loop(
                    bs_cond, bs_body, ((istart >> 4) + 1, jnp.int32(NV))
                )[0]

                head_lo = jnp.maximum(jnp.minimum(istart - LANES, istart & -64), 0)
                head_hi = jnp.minimum((istart + 63) & -64, SUB)
                gather_fix(head_lo, head_hi)

                sf = s_spl.astype(jnp.float32)

                def cseg(r):
                    return plsc.load_gather(ctab_v, [s_spl * 8 + r])

                c0x, c1x, c2x, c3x = cseg(0), cseg(1), cseg(2), cseg(3)
                c0y, c1y, c2y, c3y = cseg(4), cseg(5), cseg(6), cseg(7)
                lo64 = (istart + 63) & -64
                hi64 = jnp.maximum((vb * LANES) & -64, lo64)

                @plsc.parallel_loop(lo64, hi64, 64)
                def dense(i0):
                    for q in range(4):
                        o = i0 + q * LANES
                        tv = t_v[pl.ds(o, LANES)]
                        u = jnp.minimum(tv, tmax) * nseg_f
                        pt = u - sf
                        x = ((c3x * pt + c2x) * pt + c1x) * pt + c0x
                        y = ((c3y * pt + c2y) * pt + c1y) * pt + c0y
                        x_v[pl.ds(o, LANES)] = x
                        y_v[pl.ds(o, LANES)] = y

                return vb * LANES

            lax.while_loop(while_cond, while_body, jnp.int32(0))
            gather_fix(SUB - LANES, SUB)

        for j in range(NB):
            b = j % 2
            in_copy(j, b).wait()
            if j + 1 < NB:
                in_copy(j + 1, 1 - b).start()
            if j >= 2:
                out_copy(j - 2, b, 0, x_bufs[b]).wait()
                out_copy(j - 2, b, 1, y_bufs[b]).wait()
            compute(t_bufs[b], x_bufs[b], y_bufs[b])
            out_copy(j, b, 0, x_bufs[b]).start()
            out_copy(j, b, 1, y_bufs[b]).start()
        for j in (NB - 2, NB - 1):
            b = j % 2
            out_copy(j, b, 0, x_bufs[b]).wait()
            out_copy(j, b, 1, y_bufs[b]).wait()

    return k


def kernel(t, control_points, joint_points):
    ns = joint_points.shape[0] - 1
    n_t = t.shape[0]
    out = _spline_sc(ns, n_t)(
        t, control_points.reshape(-1), joint_points.reshape(-1)
    )
    return out.T


# speedup vs baseline: 1.6169x; 1.0452x over previous
import functools

import jax
import jax.numpy as jnp
import numpy as np
from jax import lax
from jax.experimental import pallas as pl
from jax.experimental.pallas import tpu as pltpu
from jax.experimental.pallas import tpu_sc as plsc

DEGREE = 3
LANES = 16


@functools.cache
def _spline_sc(num_segments, n_t):
    info = plsc.get_sparse_core_info()
    nc, ns = info.num_cores, info.num_subcores
    nw = nc * ns
    chunk = n_t // nw
    NB = 2
    SUB = chunk // NB
    n_cp = num_segments * (DEGREE - 1) * 2
    n_jp = (num_segments + 1) * 2

    @functools.partial(
        pl.kernel,
        out_type=jax.ShapeDtypeStruct((2, n_t), jnp.float32),
        mesh=plsc.VectorSubcoreMesh(core_axis_name="c", subcore_axis_name="s"),
        compiler_params=pltpu.CompilerParams(needs_layout_passes=False),
        scratch_types=[
            pltpu.VMEM((SUB,), jnp.float32),
            pltpu.VMEM((SUB,), jnp.float32),
            pltpu.VMEM((SUB,), jnp.float32),
            pltpu.VMEM((SUB,), jnp.float32),
            pltpu.VMEM((SUB,), jnp.float32),
            pltpu.VMEM((SUB,), jnp.float32),
            pltpu.VMEM((8 * num_segments + LANES,), jnp.float32),
            pltpu.VMEM((num_segments,), jnp.int32),
            pltpu.VMEM((n_cp,), jnp.float32),
            pltpu.VMEM((n_jp,), jnp.float32),
            pltpu.SemaphoreType.DMA,
            pltpu.SemaphoreType.DMA,
            pltpu.SemaphoreType.DMA,
            pltpu.SemaphoreType.DMA,
        ],
    )
    def k(t_hbm, cp_hbm, jp_hbm, out_hbm,
          t0_v, t1_v, x0_v, x1_v, y0_v, y1_v, ctab_v, thr_v, cp_v, jp_v,
          in0_s, in1_s, out0_s, out1_s):
        wid = lax.axis_index("s") * nc + lax.axis_index("c")
        base = wid * chunk
        t_bufs = (t0_v, t1_v)
        x_bufs = (x0_v, x1_v)
        y_bufs = (y0_v, y1_v)
        in_sems = (in0_s, in1_s)
        out_sems = (out0_s, out1_s)

        def in_copy(j, b):
            return pltpu.make_async_copy(
                t_hbm.at[pl.ds(base + j * SUB, SUB)], t_bufs[b], in_sems[b]
            )

        def out_copy(j, b, row, buf):
            return pltpu.make_async_copy(
                buf, out_hbm.at[row, pl.ds(base + j * SUB, SUB)], out_sems[b]
            )

        in_copy(0, 0).start()
        pltpu.sync_copy(cp_hbm, cp_v)
        pltpu.sync_copy(jp_hbm, jp_v)
        iv = lax.iota(jnp.int32, LANES)
        nseg_f = float(num_segments)
        inv_nseg = 1.0 / num_segments
        tmax = float(np.nextafter(np.float32(1.0), np.float32(0.0)))

        @plsc.parallel_loop(0, num_segments, LANES)
        def build(s0):
            segv = s0 + iv
            j2 = segv * 2
            c4 = segv * 4
            s8 = segv * 8

            def jp(off):
                return plsc.load_gather(jp_v, [j2 + off])

            def cp(off):
                return plsc.load_gather(cp_v, [c4 + off])

            for d in range(2):
                p0 = jp(d)
                p1 = cp(d)
                p2 = cp(2 + d)
                p3 = jp(2 + d)
                c1 = 3.0 * (p1 - p0)
                c2 = 3.0 * ((p0 - p1) + (p2 - p1))
                c3 = (p3 - p0) + 3.0 * (p1 - p2)
                plsc.store_scatter(ctab_v, [s8 + 4 * d], p0)
                plsc.store_scatter(ctab_v, [s8 + 4 * d + 1], c1)
                plsc.store_scatter(ctab_v, [s8 + 4 * d + 2], c2)
                plsc.store_scatter(ctab_v, [s8 + 4 * d + 3], c3)
            thr_f = (segv + 1).astype(jnp.float32) * inv_nseg
            plsc.store_scatter(thr_v, [segv], plsc.bitcast(thr_f, jnp.int32))

        NV = SUB // LANES

        def compute(t_v, x_v, y_v):
            def gather_fix(lo, hi):
                @plsc.parallel_loop(lo, hi, LANES)
                def fix(o):
                    tv = t_v[pl.ds(o, LANES)]
                    u = jnp.minimum(tv, tmax) * nseg_f
                    seg = u.astype(jnp.int32)
                    pt = u - seg.astype(jnp.float32)
                    seg8 = seg << 3

                    def g(r):
                        return plsc.load_gather(ctab_v, [seg8 + r])

                    x = ((g(3) * pt + g(2)) * pt + g(1)) * pt + g(0)
                    y = ((g(7) * pt + g(6)) * pt + g(5)) * pt + g(4)
                    x_v[pl.ds(o, LANES)] = x
                    y_v[pl.ds(o, LANES)] = y

            def lane0_bits(i16):
                v = t_v[pl.ds(i16, LANES)]
                return lax.bitcast_convert_type(v[0], jnp.int32)

            def while_cond(istart):
                return istart < SUB

            def while_body(istart):
                tv0 = t_v[pl.ds(istart, LANES)]
                u0 = jnp.minimum(tv0, tmax) * nseg_f
                seg0 = u0.astype(jnp.int32)
                s_spl = plsc.cummax(
                    jnp.where(iv == 0, seg0, jnp.int32(-2147483648))
                )
                thr_b = plsc.load_gather(thr_v, [s_spl])[0]

                def bs_cond(c):
                    return c[0] < c[1]

                def bs_body(c):
                    lo, hi = c
                    mid = (lo + hi) >> 1
                    pred = lane0_bits(mid * LANES) < thr_b
                    return (jnp.where(pred, mid + 1, lo),
                            jnp.where(pred, hi, mid))

                vb = lax.while_loop(
                    bs_cond, bs_body, ((istart >> 4) + 1, jnp.int32(NV))
                )[0]

                head_lo = jnp.maximum(jnp.minimum(istart - LANES, istart & -64), 0)
                head_hi = jnp.minimum((istart + 63) & -64, SUB)
                gather_fix(head_lo, head_hi)

                sf = s_spl.astype(jnp.float32)

                def cseg(r):
                    return plsc.load_gather(ctab_v, [s_spl * 8 + r])

                c0x, c1x, c2x, c3x = cseg(0), cseg(1), cseg(2), cseg(3)
                c0y, c1y, c2y, c3y = cseg(4), cseg(5), cseg(6), cseg(7)
                lo64 = (istart + 63) & -64
                hi64 = jnp.maximum((vb * LANES) & -64, lo64)

                @plsc.parallel_loop(lo64, hi64, 64)
                def dense(i0):
                    for q in range(4):
                        o = i0 + q * LANES
                        tv = t_v[pl.ds(o, LANES)]
                        u = jnp.minimum(tv, tmax) * nseg_f
                        pt = u - sf
                        x = ((c3x * pt + c2x) * pt + c1x) * pt + c0x
                        y = ((c3y * pt + c2y) * pt + c1y) * pt + c0y
                        x_v[pl.ds(o, LANES)] = x
                        y_v[pl.ds(o, LANES)] = y

                return vb * LANES

            lax.while_loop(while_cond, while_body, jnp.int32(0))
            gather_fix(SUB - LANES, SUB)

        for j in range(NB):
            b = j % 2
            in_copy(j, b).wait()
            if j + 1 < NB:
                in_copy(j + 1, 1 - b).start()
            if j >= 2:
                out_copy(j - 2, b, 0, x_bufs[b]).wait()
                out_copy(j - 2, b, 1, y_bufs[b]).wait()
            compute(t_bufs[b], x_bufs[b], y_bufs[b])
            out_copy(j, b, 0, x_bufs[b]).start()
            out_copy(j, b, 1, y_bufs[b]).start()
        for j in (NB - 2, NB - 1):
            b = j % 2
            out_copy(j, b, 0, x_bufs[b]).wait()
            out_copy(j, b, 1, y_bufs[b]).wait()

    return k


def kernel(t, control_points, joint_points):
    ns = joint_points.shape[0] - 1
    n_t = t.shape[0]
    out = _spline_sc(ns, n_t)(
        t, control_points.reshape(-1), joint_points.reshape(-1)
    )
    return out.T
